# baseline (speedup 1.0000x reference)
import dataclasses
import functools

import jax
import jax.numpy as jnp
from jax import lax
from jax.experimental import pallas as pl
from jax.experimental.pallas import tpu as pltpu
from jax.experimental.pallas import tpu_sc as plsc

N = 10000
E = 320000
D = 128
K_RANK = 1000

NC = 2
NS = 16
NW = NC * NS
PER_TILE = E // NW
P1_ITERS = PER_TILE // 16
CH2 = 96
CMP_CAP = PER_TILE + CH2
NRING = 2
PACKED = 1264
NPAD = 10112
ROWS_PER_SUB = NPAD // NS


def _sc_scatter_build():
    mesh = plsc.VectorSubcoreMesh(core_axis_name="c", subcore_axis_name="s")
    cp = pltpu.CompilerParams()
    if "needs_layout_passes" in pltpu.CompilerParams.__dataclass_fields__:
        cp = dataclasses.replace(cp, needs_layout_passes=False)

    @functools.partial(
        pl.kernel,
        out_type=jax.ShapeDtypeStruct((NC, NPAD, D), jnp.float32),
        mesh=mesh,
        compiler_params=cp,
        scratch_types=(
            [pltpu.VMEM((CMP_CAP,), jnp.int32),
             pltpu.VMEM((CMP_CAP,), jnp.int32),
             pltpu.VMEM((PACKED,), jnp.int32)]
            + [pltpu.VMEM((CH2,), jnp.int32) for _ in range(2 * NRING)]
            + [pltpu.VMEM((CH2, D), jnp.float32) for _ in range(NRING)]
            + [pltpu.VMEM_SHARED((NPAD, D), jnp.float32)]
            + [pltpu.SemaphoreType.DMA for _ in range(3 + 2 * NRING)]
        ),
    )
    def sc_kernel(h_hbm, edge_hbm, pmask_hbm, zeros_hbm, out_hbm,
                  rows_buf, cols_buf, pmask_buf, *rest):
        row_sc = rest[0:2 * NRING:2]
        col_sc = rest[1:2 * NRING:2]
        gath = rest[2 * NRING:3 * NRING]
        acc = rest[3 * NRING]
        sems = rest[3 * NRING + 1:]
        sem_e0, sem_e1, sem_m = sems[0:3]
        sem_g = sems[3:3 + NRING]
        sem_s = sems[3 + NRING:]

        c = lax.axis_index("c")
        s = lax.axis_index("s")
        wid = c * NS + s
        ebase = pl.multiple_of(wid * PER_TILE, 8)
        rbase = pl.multiple_of(s * ROWS_PER_SUB, 8)

        scope = jax.named_scope
        cp_r = pltpu.async_copy(edge_hbm.at[pl.ds(ebase, PER_TILE)],
                                rows_buf.at[pl.ds(0, PER_TILE)], sem_e0)
        cp_c = pltpu.async_copy(edge_hbm.at[pl.ds(E + ebase, PER_TILE)],
                                cols_buf.at[pl.ds(0, PER_TILE)], sem_e1)
        cp_m = pltpu.async_copy(pmask_hbm, pmask_buf, sem_m)
        pltpu.sync_copy(zeros_hbm, acc.at[pl.ds(rbase, ROWS_PER_SUB)])
        with scope("stage_wait"):
            cp_r.wait()
            cp_c.wait()
            cp_m.wait()

        def p1_group(g, off):
            col16 = cols_buf[pl.ds(pl.multiple_of(g * 16, 16), 16)]
            row16 = rows_buf[pl.ds(pl.multiple_of(g * 16, 16), 16)]
            w16 = plsc.load_gather(pmask_buf, [lax.shift_right_logical(col16, 3)])
            m = (lax.shift_right_logical(w16, col16 & 7) & 1) != 0
            mi = m.astype(jnp.int32)
            dst = off + plsc.cumsum(mi) - 1
            plsc.store_scatter(cols_buf, [dst], col16, mask=m)
            plsc.store_scatter(rows_buf, [dst], row16, mask=m)
            return off + jnp.sum(mi)

        def p1(i, off):
            off = p1_group(2 * i, off)
            return p1_group(2 * i + 1, off)

        with scope("compact"):
            off = lax.fori_loop(0, P1_ITERS // 2, p1, jnp.int32(0))
            off = p1_group(P1_ITERS - 1, off)

        iota16 = lax.iota(jnp.int32, 16)
        dummy_r = N + iota16
        zero_c = jnp.zeros((16,), jnp.int32)
        ones = jnp.full((16,), True)
        for t in range(CH2 // 16):
            dst = off + t * 16 + iota16
            plsc.store_scatter(cols_buf, [dst], zero_c, mask=ones)
            plsc.store_scatter(rows_buf, [dst], dummy_r, mask=ones)
        nbat = (off + CH2 - 1) // CH2

        with scope("barrier1"):
            plsc.subcore_barrier()

        scope3 = jax.named_scope("gather_scatter")
        scope3.__enter__()

        @pl.loop(0, nbat, step=NRING)
        def _(k0):
            for b in range(NRING):
                @pl.when(k0 + b < nbat)
                def _():
                    @pl.when(k0 > 0)
                    def _():
                        pltpu.make_async_copy(
                            gath[b], acc.at[row_sc[b]], sem_s[b]).wait()
                    kb = pl.multiple_of((k0 + b) * CH2, CH2)
                    for i in range(CH2 // 16):
                        col_sc[b][pl.ds(i * 16, 16)] = (
                            cols_buf[pl.ds(kb + i * 16, 16)])
                        row_sc[b][pl.ds(i * 16, 16)] = (
                            rows_buf[pl.ds(kb + i * 16, 16)])
                    pltpu.async_copy(h_hbm.at[col_sc[b]], gath[b], sem_g[b])
            for b in range(NRING):
                @pl.when(k0 + b < nbat)
                def _():
                    pltpu.make_async_copy(h_hbm.at[col_sc[b]], gath[b],
                                          sem_g[b]).wait()
                    pltpu.async_copy(gath[b], acc.at[row_sc[b]], sem_s[b],
                                     add=True)

        scope3.__exit__(None, None, None)
        scope2 = jax.named_scope("drain")
        scope2.__enter__()
        for b in range(NRING):
            @pl.when(nbat > b)
            def _():
                pltpu.make_async_copy(gath[b], acc.at[row_sc[b]],
                                      sem_s[b]).wait()

        scope2.__exit__(None, None, None)
        with scope("barrier2"):
            plsc.subcore_barrier()
        with scope("writeback"):
            pltpu.sync_copy(acc.at[pl.ds(rbase, ROWS_PER_SUB)],
                            out_hbm.at[c, pl.ds(rbase, ROWS_PER_SUB)])

    return sc_kernel


_sc_scatter = _sc_scatter_build()


def _linear_mask_body(x_ref, nr_ref, nr8_ref, w_ref, b_ref, o_ref, p_ref):
    h = lax.dot_general(
        x_ref[...], w_ref[...],
        dimension_numbers=(((1,), (1,)), ((), ())),
        preferred_element_type=jnp.float32,
    )
    h = h + b_ref[...]
    m = (nr_ref[...] <= K_RANK).astype(jnp.float32)
    o_ref[...] = h * m
    bits = (nr8_ref[...] <= K_RANK).astype(jnp.int32)
    sh = lax.broadcasted_iota(jnp.int32, (N // 8, 8), 1)
    p_ref[...] = jnp.sum(bits << sh, axis=1, keepdims=True)


def _linear_mask(x, nr_col, nr8, W, b_row):
    return pl.pallas_call(
        _linear_mask_body,
        out_shape=(jax.ShapeDtypeStruct((N, D), jnp.float32),
                   jax.ShapeDtypeStruct((N // 8, 1), jnp.int32)),
    )(x, nr_col, nr8, W, b_row)


def _combine_body(p_ref, o_ref):
    o_ref[...] = p_ref[0] + p_ref[1]


def _combine(partial):
    blk = 2000
    return pl.pallas_call(
        _combine_body,
        grid=(N // blk,),
        in_specs=[pl.BlockSpec((NC, blk, D), lambda i: (0, i, 0))],
        out_specs=pl.BlockSpec((blk, D), lambda i: (i, 0)),
        out_shape=jax.ShapeDtypeStruct((N, D), jnp.float32),
    )(partial)


def kernel(x, edge_index, node_rankings, W, b):
    zeros = jnp.zeros((ROWS_PER_SUB, D), jnp.float32)
    nr_col = node_rankings.reshape(N, 1)
    nr8 = node_rankings.reshape(N // 8, 8)
    b_row = b.reshape(1, D)

    h, packed = _linear_mask(x, nr_col, nr8, W, b_row)
    packed_p = jnp.pad(packed.reshape(N // 8), (0, PACKED - N // 8))
    partial = _sc_scatter(h, edge_index.reshape(2 * E), packed_p, zeros)
    return _combine(partial)


# speedup vs baseline: 1.1573x; 1.1573x over previous
import dataclasses
import functools

import jax
import jax.numpy as jnp
from jax import lax
from jax.experimental import pallas as pl
from jax.experimental.pallas import tpu as pltpu
from jax.experimental.pallas import tpu_sc as plsc

N = 10000
E = 320000
D = 128
K_RANK = 1000

NC = 2
NS = 16
NW = NC * NS
PER_TILE = E // NW
P1_ITERS = PER_TILE // 16
CH2 = 64
CMP_CAP = PER_TILE + CH2
NRING = 2
PACKED = 1264
NPAD = 10112
ROWS_PER_SUB = NPAD // NS


def _sc_scatter_build():
    mesh = plsc.VectorSubcoreMesh(core_axis_name="c", subcore_axis_name="s")
    cp = pltpu.CompilerParams()
    if "needs_layout_passes" in pltpu.CompilerParams.__dataclass_fields__:
        cp = dataclasses.replace(cp, needs_layout_passes=False)

    @functools.partial(
        pl.kernel,
        out_type=jax.ShapeDtypeStruct((NC, NPAD, D), jnp.float32),
        mesh=mesh,
        compiler_params=cp,
        scratch_types=(
            [pltpu.VMEM((CMP_CAP,), jnp.int32),
             pltpu.VMEM((CMP_CAP,), jnp.int32),
             pltpu.VMEM((N,), jnp.int32)]
            + [pltpu.VMEM((CH2,), jnp.int32) for _ in range(2 * NRING)]
            + [pltpu.VMEM((CH2, D), jnp.float32) for _ in range(NRING)]
            + [pltpu.VMEM_SHARED((NPAD, D), jnp.float32)]
            + [pltpu.SemaphoreType.DMA for _ in range(3 + 2 * NRING)]
        ),
    )
    def sc_kernel(h_hbm, edge_hbm, rank_hbm, zeros_hbm, out_hbm,
                  rows_buf, cols_buf, rank_buf, *rest):
        row_sc = rest[0:2 * NRING:2]
        col_sc = rest[1:2 * NRING:2]
        gath = rest[2 * NRING:3 * NRING]
        acc = rest[3 * NRING]
        sems = rest[3 * NRING + 1:]
        sem_e0, sem_e1, sem_m = sems[0:3]
        sem_g = sems[3:3 + NRING]
        sem_s = sems[3 + NRING:]

        c = lax.axis_index("c")
        s = lax.axis_index("s")
        wid = c * NS + s
        ebase = pl.multiple_of(wid * PER_TILE, 8)
        rbase = pl.multiple_of(s * ROWS_PER_SUB, 8)

        scope = jax.named_scope
        cp_r = pltpu.async_copy(edge_hbm.at[pl.ds(ebase, PER_TILE)],
                                rows_buf.at[pl.ds(0, PER_TILE)], sem_e0)
        cp_c = pltpu.async_copy(edge_hbm.at[pl.ds(E + ebase, PER_TILE)],
                                cols_buf.at[pl.ds(0, PER_TILE)], sem_e1)
        cp_m = pltpu.async_copy(rank_hbm, rank_buf, sem_m)
        pltpu.sync_copy(zeros_hbm, acc.at[pl.ds(rbase, ROWS_PER_SUB)])
        with scope("stage_wait"):
            cp_r.wait()
            cp_c.wait()
            cp_m.wait()

        def p1_group(g, off):
            col16 = cols_buf[pl.ds(pl.multiple_of(g * 16, 16), 16)]
            row16 = rows_buf[pl.ds(pl.multiple_of(g * 16, 16), 16)]
            rk = plsc.load_gather(rank_buf, [col16])
            m = rk <= K_RANK
            mi = m.astype(jnp.int32)
            dst = off + plsc.cumsum(mi) - 1
            plsc.store_scatter(cols_buf, [dst], col16, mask=m)
            plsc.store_scatter(rows_buf, [dst], row16, mask=m)
            return off + jnp.sum(mi)

        def p1(i, off):
            off = p1_group(2 * i, off)
            return p1_group(2 * i + 1, off)

        with scope("compact"):
            off = lax.fori_loop(0, P1_ITERS // 2, p1, jnp.int32(0))
            off = p1_group(P1_ITERS - 1, off)

        iota16 = lax.iota(jnp.int32, 16)
        dummy_r = N + iota16
        zero_c = jnp.zeros((16,), jnp.int32)
        ones = jnp.full((16,), True)
        for t in range(CH2 // 16):
            dst = off + t * 16 + iota16
            plsc.store_scatter(cols_buf, [dst], zero_c, mask=ones)
            plsc.store_scatter(rows_buf, [dst], dummy_r, mask=ones)
        nbat = (off + CH2 - 1) // CH2

        with scope("barrier1"):
            plsc.subcore_barrier()

        scope3 = jax.named_scope("gather_scatter")
        scope3.__enter__()

        @pl.loop(0, nbat, step=NRING)
        def _(k0):
            for b in range(NRING):
                @pl.when(k0 + b < nbat)
                def _():
                    @pl.when(k0 > 0)
                    def _():
                        pltpu.make_async_copy(
                            gath[b], acc.at[row_sc[b]], sem_s[b]).wait()
                    kb = pl.multiple_of((k0 + b) * CH2, CH2)
                    for i in range(CH2 // 16):
                        col_sc[b][pl.ds(i * 16, 16)] = (
                            cols_buf[pl.ds(kb + i * 16, 16)])
                        row_sc[b][pl.ds(i * 16, 16)] = (
                            rows_buf[pl.ds(kb + i * 16, 16)])
                    pltpu.async_copy(h_hbm.at[col_sc[b]], gath[b], sem_g[b])
            for b in range(NRING):
                @pl.when(k0 + b < nbat)
                def _():
                    pltpu.make_async_copy(h_hbm.at[col_sc[b]], gath[b],
                                          sem_g[b]).wait()
                    pltpu.async_copy(gath[b], acc.at[row_sc[b]], sem_s[b],
                                     add=True)

        scope3.__exit__(None, None, None)
        scope2 = jax.named_scope("drain")
        scope2.__enter__()
        for b in range(NRING):
            @pl.when(nbat > b)
            def _():
                pltpu.make_async_copy(gath[b], acc.at[row_sc[b]],
                                      sem_s[b]).wait()

        scope2.__exit__(None, None, None)
        with scope("barrier2"):
            plsc.subcore_barrier()
        with scope("writeback"):
            pltpu.sync_copy(acc.at[pl.ds(rbase, ROWS_PER_SUB)],
                            out_hbm.at[c, pl.ds(rbase, ROWS_PER_SUB)])

    return sc_kernel


_sc_scatter = _sc_scatter_build()


def _linear_mask_body(x_ref, nr_ref, w_ref, b_ref, o_ref):
    h = lax.dot_general(
        x_ref[...], w_ref[...],
        dimension_numbers=(((1,), (1,)), ((), ())),
        preferred_element_type=jnp.float32,
    )
    h = h + b_ref[...]
    m = (nr_ref[...] <= K_RANK).astype(jnp.float32)
    o_ref[...] = h * m


def _linear_mask(x, nr_col, W, b_row):
    return pl.pallas_call(
        _linear_mask_body,
        out_shape=jax.ShapeDtypeStruct((N, D), jnp.float32),
    )(x, nr_col, W, b_row)


def _combine_body(p_ref, o_ref):
    o_ref[...] = p_ref[0] + p_ref[1]


def _combine(partial):
    blk = 2000
    return pl.pallas_call(
        _combine_body,
        grid=(N // blk,),
        in_specs=[pl.BlockSpec((NC, blk, D), lambda i: (0, i, 0))],
        out_specs=pl.BlockSpec((blk, D), lambda i: (i, 0)),
        out_shape=jax.ShapeDtypeStruct((N, D), jnp.float32),
    )(partial)


def kernel(x, edge_index, node_rankings, W, b):
    zeros = jnp.zeros((ROWS_PER_SUB, D), jnp.float32)
    nr_col = node_rankings.reshape(N, 1)
    b_row = b.reshape(1, D)

    h = _linear_mask(x, nr_col, W, b_row)
    partial = _sc_scatter(h, edge_index.reshape(2 * E),
                          node_rankings.reshape(N), zeros)
    return _combine(partial)
